# baseline (speedup 1.0000x reference)
import jax
import jax.numpy as jnp
from jax.experimental import pallas as pl
from jax.experimental.pallas import tpu as pltpu

BATCH = 8
N_TILES = 4
N_TOKENS = 1025
EMBED_DIM = 1280
MAX_NUM_TILES = 4

ITEMS = BATCH * N_TILES
NSLOTS = 4


def _coords(k):
    b = k // N_TILES
    t = k - b * N_TILES
    as_i32 = lambda v: jnp.asarray(v, dtype=jnp.int32)
    return as_i32(b), as_i32(t)


def _in_copy(k, slot, x_ref, in_buf, in_sems):
    b, t = _coords(k)
    slot = jnp.asarray(slot, dtype=jnp.int32)
    return pltpu.make_async_copy(
        x_ref.at[b, t], in_buf.at[slot], in_sems.at[slot])


def _out_copy(k, slot, out_ref, out_buf, out_sems):
    b, t = _coords(k)
    slot = jnp.asarray(slot, dtype=jnp.int32)
    return pltpu.make_async_copy(
        out_buf.at[slot], out_ref.at[b, t], out_sems.at[slot])


def _body(ar_ref, gate_ref, x_ref, emb_ref, out_ref,
          in_buf, out_buf, in_sems, out_sems):
    for s in range(NSLOTS):
        _in_copy(s, s, x_ref, in_buf, in_sems).start()

    gate_t = jnp.tanh(gate_ref[0])

    def loop(k, carry):
        slot = jax.lax.rem(k, jnp.int32(NSLOTS))
        _in_copy(k, slot, x_ref, in_buf, in_sems).wait()

        b, t = _coords(k)
        h = ar_ref[2 * b]
        w = ar_ref[2 * b + 1]
        valid = t < h * w
        w_safe = jnp.maximum(w, 1)
        i = jnp.where(valid, t // w_safe, 0)
        j = jnp.where(valid, t % w_safe, 0)
        row = emb_ref[i, j]
        coef = jnp.where(valid, gate_t, 0.0)

        @pl.when(k >= NSLOTS)
        def _():
            _out_copy(k - NSLOTS, slot, out_ref, out_buf, out_sems).wait()

        out_buf[slot] = in_buf[slot] + coef * row

        _out_copy(k, slot, out_ref, out_buf, out_sems).start()

        @pl.when(k + NSLOTS < ITEMS)
        def _():
            _in_copy(k + NSLOTS, slot, x_ref, in_buf, in_sems).start()

        return carry

    jax.lax.fori_loop(jnp.int32(0), jnp.int32(ITEMS), loop, jnp.int32(0))

    for s in range(NSLOTS):
        k = ITEMS - NSLOTS + s
        _out_copy(k, k % NSLOTS, out_ref, out_buf, out_sems).wait()


def kernel(x, aspect_ratio, embedding, gate):
    ar = aspect_ratio.astype(jnp.int32).reshape(-1)

    grid_spec = pltpu.PrefetchScalarGridSpec(
        num_scalar_prefetch=2,
        grid=(1, 1),
        in_specs=[
            pl.BlockSpec(memory_space=pl.ANY),
            pl.BlockSpec((MAX_NUM_TILES, MAX_NUM_TILES, 1, EMBED_DIM),
                         lambda z0, z1, ar, g: (z0, z1, z0, z1)),
        ],
        out_specs=pl.BlockSpec(memory_space=pl.ANY),
        scratch_shapes=[
            pltpu.VMEM((NSLOTS, N_TOKENS, EMBED_DIM), jnp.float32),
            pltpu.VMEM((NSLOTS, N_TOKENS, EMBED_DIM), jnp.float32),
            pltpu.SemaphoreType.DMA((NSLOTS,)),
            pltpu.SemaphoreType.DMA((NSLOTS,)),
        ],
    )

    out = pl.pallas_call(
        _body,
        grid_spec=grid_spec,
        out_shape=jax.ShapeDtypeStruct(x.shape, x.dtype),
    )(ar, gate.astype(jnp.float32), x, embedding)
    return out


# speedup vs baseline: 1.0004x; 1.0004x over previous
import jax
import jax.numpy as jnp
from jax.experimental import pallas as pl
from jax.experimental.pallas import tpu as pltpu

BATCH = 8
N_TILES = 4
N_TOKENS = 1025
EMBED_DIM = 1280
MAX_NUM_TILES = 4

ITEMS = BATCH * N_TILES
NSLOTS = 4


def _coords(k):
    b = k // N_TILES
    t = k - b * N_TILES
    as_i32 = lambda v: jnp.asarray(v, dtype=jnp.int32)
    return as_i32(b), as_i32(t)


def _in_copy(k, x_ref, in_buf, in_sem):
    b, t = _coords(k)
    return pltpu.make_async_copy(x_ref.at[b, t], in_buf, in_sem)


def _out_copy(k, out_ref, out_buf, out_sem):
    b, t = _coords(k)
    return pltpu.make_async_copy(out_buf, out_ref.at[b, t], out_sem)


def _body(ar_ref, gate_ref, x_ref, emb_ref, out_ref, *scratch):
    in_bufs = scratch[0:NSLOTS]
    out_bufs = scratch[NSLOTS:2 * NSLOTS]
    in_sems = scratch[2 * NSLOTS:3 * NSLOTS]
    out_sems = scratch[3 * NSLOTS:4 * NSLOTS]

    for s in range(NSLOTS):
        _in_copy(s, x_ref, in_bufs[s], in_sems[s]).start()

    gate_t = jnp.tanh(gate_ref[0])

    def make_step(s):
        def step(k):
            _in_copy(k, x_ref, in_bufs[s], in_sems[s]).wait()

            b, t = _coords(k)
            h = ar_ref[2 * b]
            w = ar_ref[2 * b + 1]
            valid = t < h * w
            w_safe = jnp.maximum(w, 1)
            i = jnp.where(valid, t // w_safe, 0)
            j = jnp.where(valid, t % w_safe, 0)
            row = emb_ref[i, j]
            coef = jnp.where(valid, gate_t, 0.0)

            @pl.when(k >= NSLOTS)
            def _():
                _out_copy(k - NSLOTS, out_ref, out_bufs[s], out_sems[s]).wait()

            out_bufs[s][...] = in_bufs[s][...] + coef * row

            _out_copy(k, out_ref, out_bufs[s], out_sems[s]).start()

            @pl.when(k + NSLOTS < ITEMS)
            def _():
                _in_copy(k + NSLOTS, x_ref, in_bufs[s], in_sems[s]).start()
        return step

    steps = [make_step(s) for s in range(NSLOTS)]

    def loop(r, carry):
        base = r * NSLOTS
        for s in range(NSLOTS):
            steps[s](base + jnp.int32(s))
        return carry

    jax.lax.fori_loop(jnp.int32(0), jnp.int32(ITEMS // NSLOTS), loop,
                      jnp.int32(0))

    for s in range(NSLOTS):
        k = ITEMS - NSLOTS + s
        _out_copy(k, out_ref, out_bufs[s], out_sems[s]).wait()


def kernel(x, aspect_ratio, embedding, gate):
    ar = aspect_ratio.astype(jnp.int32).reshape(-1)

    scratch_shapes = (
        [pltpu.VMEM((N_TOKENS, EMBED_DIM), jnp.float32) for _ in range(NSLOTS)]
        + [pltpu.VMEM((N_TOKENS, EMBED_DIM), jnp.float32) for _ in range(NSLOTS)]
        + [pltpu.SemaphoreType.DMA for _ in range(2 * NSLOTS)]
    )

    grid_spec = pltpu.PrefetchScalarGridSpec(
        num_scalar_prefetch=2,
        grid=(1, 1),
        in_specs=[
            pl.BlockSpec(memory_space=pl.ANY),
            pl.BlockSpec((MAX_NUM_TILES, MAX_NUM_TILES, 1, EMBED_DIM),
                         lambda z0, z1, ar, g: (z0, z1, z0, z1)),
        ],
        out_specs=pl.BlockSpec(memory_space=pl.ANY),
        scratch_shapes=scratch_shapes,
    )

    out = pl.pallas_call(
        _body,
        grid_spec=grid_spec,
        out_shape=jax.ShapeDtypeStruct(x.shape, x.dtype),
    )(ar, gate.astype(jnp.float32), x, embedding)
    return out
